# baseline (speedup 1.0000x reference)
import jax
import jax.numpy as jnp
from jax import lax
from jax.experimental import pallas as pl
from jax.experimental.pallas import tpu as pltpu
import jax.experimental.pallas.tpu_sc as plsc

N = 10000
D = 128
H = 128
E = 320000

NC = 2
NS = 16
K = 80
SLAB = 9
NSL = 28
CPS = SLAB * NSL
EPS = CPS * K
EPAD = EPS * NS
NPAD = N + 16
RZ = NPAD // NS
RPS = N // NS
CW = 8
NB = 3
NP = NSL // 2


def _sc_body(feat, srcA, dstA, srcB, dstB, srcC, dstC, srcD, dstD,
             zrows, zcnt, ones_h,
             sumsA, cntsA, sumsB, cntsB, sumsC, cntsC, sumsD, cntsD,
             acc, cnt, rows0, rows1, rows2, sia, dia, sib, dib, ones_v,
             gsem0, gsem1, gsem2, ssem0, ssem1, ssem2, isema, isemb):
    rows = (rows0, rows1, rows2)
    gsem = (gsem0, gsem1, gsem2)
    ssem = (ssem0, ssem1, ssem2)
    bufA = (sia, dia, isema)
    bufB = (sib, dib, isemb)
    c = lax.axis_index("c")
    s = lax.axis_index("s")

    pltpu.sync_copy(ones_h, ones_v)

    def run_list(src1d, dst1d, sums_h, cnts_h):
        pltpu.sync_copy(zrows, acc.at[pl.ds(s * RZ, RZ)])
        pltpu.sync_copy(zcnt, cnt.at[pl.ds(s * RZ, RZ)])
        base = s * EPS

        def issue_idx(tsl, buf):
            si, di, isem = buf
            off = base + tsl * (SLAB * K)
            pltpu.async_copy(src1d.at[pl.ds(off, SLAB * K)], si, isem)
            pltpu.async_copy(dst1d.at[pl.ds(off, SLAB * K)], di, isem)

        def wait_idx(buf):
            si, di, isem = buf
            pltpu.make_async_copy(src1d.at[pl.ds(base, SLAB * K)], si,
                                  isem).wait()
            pltpu.make_async_copy(dst1d.at[pl.ds(base, SLAB * K)], di,
                                  isem).wait()

        def buf_off(jj):
            j = jj % (2 * SLAB)
            return (bufA, (j % SLAB) * K) if j < SLAB else \
                   (bufB, (j % SLAB) * K)

        def wait_scatter(buf, off, b):
            _, di, _ = buf
            pltpu.make_async_copy(rows[b], acc.at[di.at[pl.ds(off, K)]],
                                  ssem[b]).wait()
            pltpu.make_async_copy(ones_v, cnt.at[di.at[pl.ds(off, K)]],
                                  ssem[b]).wait()

        def pair_body(tsl1, tsl2, first, last):
            for jj in range(2 * SLAB):
                b = jj % NB
                b2 = (jj + 2) % NB
                cur, coff = buf_off(jj)
                si_c, di_c, _ = cur
                if jj == 16 and not last:
                    wait_idx(bufA)
                pltpu.make_async_copy(
                    feat.at[si_c.at[pl.ds(coff, K)]], rows[b],
                    gsem[b]).wait()
                if jj < 16 or not last:
                    lb, loff = buf_off(jj + 2)
                    si_l = lb[0]
                    if not (first and jj == 0):
                        if jj == 0:
                            pbuf, poff = bufB, (SLAB - 1) * K
                        else:
                            pbuf, poff = buf_off(jj - 1)
                        wait_scatter(pbuf, poff, b2)
                    pltpu.async_copy(feat.at[si_l.at[pl.ds(loff, K)]],
                                     rows[b2], gsem[b2])
                pltpu.async_copy(rows[b], acc.at[di_c.at[pl.ds(coff, K)]],
                                ssem[b], add=True)
                pltpu.async_copy(ones_v, cnt.at[di_c.at[pl.ds(coff, K)]],
                                ssem[b], add=True)
                if jj == 2:
                    issue_idx(tsl1, bufB)
                if jj == 7:
                    wait_idx(bufB)
                if jj == 11 and not last:
                    issue_idx(tsl2, bufA)
            if last:
                for jd in range(2 * SLAB - NB, 2 * SLAB):
                    bd = jd % NB
                    pb, po = buf_off(jd)
                    wait_scatter(pb, po, bd)

        issue_idx(0, bufA)
        wait_idx(bufA)
        pltpu.async_copy(feat.at[sia.at[pl.ds(0, K)]], rows[0], gsem[0])
        pltpu.async_copy(feat.at[sia.at[pl.ds(K, K)]], rows[1], gsem[1])
        plsc.subcore_barrier()

        pair_body(1, 2, True, False)

        @pl.loop(1, NP - 1)
        def mid(u):
            pair_body(2 * u + 1, 2 * u + 2, False, False)

        pair_body(NSL - 1, 0, False, True)

        plsc.subcore_barrier()
        pltpu.sync_copy(acc.at[pl.ds(s * RPS, RPS)],
                        sums_h.at[pl.ds(s * RPS, RPS)])
        pltpu.sync_copy(cnt.at[pl.ds(s * RPS, RPS)],
                        cnts_h.at[pl.ds(s * RPS, RPS)])
        plsc.subcore_barrier()

    @pl.when(c == 0)
    def _():
        run_list(srcA, dstA, sumsA, cntsA)
        run_list(srcB, dstB, sumsB, cntsB)

    @pl.when(c == 1)
    def _():
        run_list(srcC, dstC, sumsC, cntsC)
        run_list(srcD, dstD, sumsD, cntsD)


_sc_aggregate = pl.kernel(
    _sc_body,
    out_type=[jax.ShapeDtypeStruct((N, D), jnp.float32),
              jax.ShapeDtypeStruct((N, CW), jnp.float32)] * 4,
    mesh=plsc.VectorSubcoreMesh(core_axis_name="c", subcore_axis_name="s"),
    compiler_params=pltpu.CompilerParams(use_tc_tiling_on_sc=False),
    scratch_types=(
        [pltpu.VMEM_SHARED((NPAD, D), jnp.float32),
         pltpu.VMEM_SHARED((NPAD, CW), jnp.float32)]
        + [pltpu.VMEM((K, D), jnp.float32)] * NB
        + [pltpu.VMEM((SLAB * K,), jnp.int32)] * 4
        + [pltpu.VMEM((K, CW), jnp.float32)]
        + [pltpu.SemaphoreType.DMA] * (2 * NB + 2)
    ),
)


def _tc_body(sa, ca, sb, cb, w1, sc_, cc_, sd, cd, w3, o_src, o_tgt):
    ma = sa[...] / jnp.maximum(ca[:, 0:1], 1.0)
    mb = sb[...] / jnp.maximum(cb[:, 0:1], 1.0)
    mc = sc_[...] / jnp.maximum(cc_[:, 0:1], 1.0)
    md = sd[...] / jnp.maximum(cd[:, 0:1], 1.0)
    f32 = jnp.float32
    s_emb = (jnp.dot(ma, w1[0:D, :], preferred_element_type=f32)
             + jnp.dot(mb, w1[D:2 * D, :], preferred_element_type=f32))
    t_emb = (jnp.dot(mc, w3[0:D, :], preferred_element_type=f32)
             + jnp.dot(md, w3[D:2 * D, :], preferred_element_type=f32))
    o_src[...] = jnp.maximum(s_emb, 0.0)
    o_tgt[...] = jnp.maximum(t_emb, 0.0)


BR = 1000


def _tc_finish(sumsA, cntsA, sumsB, cntsB, W1, sumsC, cntsC, sumsD, cntsD, W3):
    sspec = pl.BlockSpec((BR, D), lambda i: (i, 0))
    cspec = pl.BlockSpec((BR, CW), lambda i: (i, 0))
    wspec = pl.BlockSpec((2 * D, H), lambda i: (0, 0))
    return pl.pallas_call(
        _tc_body,
        grid=(N // BR,),
        in_specs=[sspec, cspec, sspec, cspec, wspec,
                  sspec, cspec, sspec, cspec, wspec],
        out_specs=[pl.BlockSpec((BR, H), lambda i: (i, 0))] * 2,
        out_shape=[jax.ShapeDtypeStruct((N, H), jnp.float32)] * 2,
    )(sumsA, cntsA, sumsB, cntsB, W1, sumsC, cntsC, sumsD, cntsD, W3)


def kernel(features, W1, W3, source_nei, target_nei, source_nei2, target_nei2):
    pad_dst = jnp.full((EPAD - E,), N, jnp.int32)
    pad_src = jnp.zeros((EPAD - E,), jnp.int32)

    def prep(nei):
        src = jnp.concatenate([nei[1], pad_src])
        dst = jnp.concatenate([nei[0], pad_dst])
        return src, dst

    srcA, dstA = prep(source_nei)
    srcB, dstB = prep(target_nei2)
    srcC, dstC = prep(target_nei)
    srcD, dstD = prep(source_nei2)

    zrows = jnp.zeros((RZ, D), jnp.float32)
    zcnt = jnp.zeros((RZ, CW), jnp.float32)
    ones_h = jnp.ones((K, CW), jnp.float32)

    (sumsA, cntsA, sumsB, cntsB,
     sumsC, cntsC, sumsD, cntsD) = _sc_aggregate(
        features, srcA, dstA, srcB, dstB, srcC, dstC, srcD, dstD,
        zrows, zcnt, ones_h)

    return tuple(_tc_finish(sumsA, cntsA, sumsB, cntsB, W1,
                            sumsC, cntsC, sumsD, cntsD, W3))


# speedup vs baseline: 1.5934x; 1.5934x over previous
import jax
import jax.numpy as jnp
from jax import lax
from jax.experimental import pallas as pl
from jax.experimental.pallas import tpu as pltpu
import jax.experimental.pallas.tpu_sc as plsc

N = 10000
D = 128
H = 128
E = 320000

NC = 2
NS = 16
K = 80
EPS = E // NS
CPS = EPS // K
RPS = N // NS
SLAB = 10
NSL = CPS // SLAB
CW = 8
NB = 3


def _sc_body(feat, srcA, dstA, srcB, dstB, srcC, dstC, srcD, dstD,
             zrows, zcnt, ones_h,
             sumsA, cntsA, sumsB, cntsB, sumsC, cntsC, sumsD, cntsD,
             acc, cnt, rows0, rows1, rows2, sia, dia, sib, dib, ones_v,
             gsem0, gsem1, gsem2, ssem0, ssem1, ssem2, isema, isemb):
    rows = (rows0, rows1, rows2)
    gsem = (gsem0, gsem1, gsem2)
    ssem = (ssem0, ssem1, ssem2)
    slabs = ((sia, dia, isema), (sib, dib, isemb))
    c = lax.axis_index("c")
    s = lax.axis_index("s")

    pltpu.sync_copy(ones_h, ones_v)

    def run_list(src1d, dst1d, sums_h, cnts_h):
        pltpu.sync_copy(zrows, acc.at[pl.ds(s * RPS, RPS)])
        pltpu.sync_copy(zcnt, cnt.at[pl.ds(s * RPS, RPS)])
        base = s * EPS

        def issue_idx(t, buf):
            si, di, isem = buf
            off = base + t * (SLAB * K)
            pltpu.async_copy(src1d.at[pl.ds(off, SLAB * K)], si, isem)
            pltpu.async_copy(dst1d.at[pl.ds(off, SLAB * K)], di, isem)

        def wait_idx(buf):
            si, di, isem = buf
            pltpu.make_async_copy(src1d.at[pl.ds(base, SLAB * K)], si,
                                  isem).wait()
            pltpu.make_async_copy(dst1d.at[pl.ds(base, SLAB * K)], di,
                                  isem).wait()

        def slab_body(t, cur, nxt, last):
            si_v, di_v, _ = cur
            wait_idx(cur)
            if not last:
                issue_idx(t + 1, nxt)
            for jp in range(NB - 1):
                pltpu.async_copy(feat.at[si_v.at[pl.ds(jp * K, K)]],
                                 rows[jp], gsem[jp])
            for j in range(SLAB):
                b = j % NB
                sl = pl.ds(j * K, K)
                pltpu.make_async_copy(feat.at[si_v.at[sl]], rows[b],
                                      gsem[b]).wait()
                if j + NB - 1 < SLAB:
                    b2 = (j + NB - 1) % NB
                    sl2 = pl.ds((j + NB - 1) * K, K)
                    if j >= 1:
                        slp = pl.ds((j - 1) * K, K)
                        pltpu.make_async_copy(rows[b2],
                                              acc.at[di_v.at[slp]],
                                              ssem[b2]).wait()
                        pltpu.make_async_copy(ones_v, cnt.at[di_v.at[slp]],
                                              ssem[b2]).wait()
                    pltpu.async_copy(feat.at[si_v.at[sl2]], rows[b2],
                                     gsem[b2])
                pltpu.async_copy(rows[b], acc.at[di_v.at[sl]], ssem[b],
                                add=True)
                pltpu.async_copy(ones_v, cnt.at[di_v.at[sl]], ssem[b],
                                add=True)
            for jd in range(SLAB - NB, SLAB):
                bd = jd % NB
                sld = pl.ds(jd * K, K)
                pltpu.make_async_copy(rows[bd], acc.at[di_v.at[sld]],
                                      ssem[bd]).wait()
                pltpu.make_async_copy(ones_v, cnt.at[di_v.at[sld]],
                                      ssem[bd]).wait()

        issue_idx(0, slabs[0])
        plsc.subcore_barrier()

        @pl.loop(0, NSL - 1, step=2)
        def pair(t0):
            slab_body(t0, slabs[0], slabs[1], False)
            slab_body(t0 + 1, slabs[1], slabs[0], False)

        slab_body(NSL - 1, slabs[0], slabs[1], True)

        plsc.subcore_barrier()
        pltpu.sync_copy(acc.at[pl.ds(s * RPS, RPS)],
                        sums_h.at[pl.ds(s * RPS, RPS)])
        pltpu.sync_copy(cnt.at[pl.ds(s * RPS, RPS)],
                        cnts_h.at[pl.ds(s * RPS, RPS)])
        plsc.subcore_barrier()

    @pl.when(c == 0)
    def _():
        run_list(srcA, dstA, sumsA, cntsA)
        run_list(srcB, dstB, sumsB, cntsB)

    @pl.when(c == 1)
    def _():
        run_list(srcC, dstC, sumsC, cntsC)
        run_list(srcD, dstD, sumsD, cntsD)


_sc_aggregate = pl.kernel(
    _sc_body,
    out_type=[jax.ShapeDtypeStruct((N, D), jnp.float32),
              jax.ShapeDtypeStruct((N, CW), jnp.float32)] * 4,
    mesh=plsc.VectorSubcoreMesh(core_axis_name="c", subcore_axis_name="s"),
    compiler_params=pltpu.CompilerParams(use_tc_tiling_on_sc=False),
    scratch_types=(
        [pltpu.VMEM_SHARED((N, D), jnp.float32),
         pltpu.VMEM_SHARED((N, CW), jnp.float32)]
        + [pltpu.VMEM((K, D), jnp.float32)] * NB
        + [pltpu.VMEM((SLAB * K,), jnp.int32)] * 4
        + [pltpu.VMEM((K, CW), jnp.float32)]
        + [pltpu.SemaphoreType.DMA] * (2 * NB + 2)
    ),
)


def _tc_body(sa, ca, sb, cb, w1, sc_, cc_, sd, cd, w3, o_src, o_tgt):
    ma = sa[...] / jnp.maximum(ca[:, 0:1], 1.0)
    mb = sb[...] / jnp.maximum(cb[:, 0:1], 1.0)
    mc = sc_[...] / jnp.maximum(cc_[:, 0:1], 1.0)
    md = sd[...] / jnp.maximum(cd[:, 0:1], 1.0)
    f32 = jnp.float32
    s_emb = (jnp.dot(ma, w1[0:D, :], preferred_element_type=f32)
             + jnp.dot(mb, w1[D:2 * D, :], preferred_element_type=f32))
    t_emb = (jnp.dot(mc, w3[0:D, :], preferred_element_type=f32)
             + jnp.dot(md, w3[D:2 * D, :], preferred_element_type=f32))
    o_src[...] = jnp.maximum(s_emb, 0.0)
    o_tgt[...] = jnp.maximum(t_emb, 0.0)


BR = 1000


def _tc_finish(sumsA, cntsA, sumsB, cntsB, W1, sumsC, cntsC, sumsD, cntsD, W3):
    sspec = pl.BlockSpec((BR, D), lambda i: (i, 0))
    cspec = pl.BlockSpec((BR, CW), lambda i: (i, 0))
    wspec = pl.BlockSpec((2 * D, H), lambda i: (0, 0))
    return pl.pallas_call(
        _tc_body,
        grid=(N // BR,),
        in_specs=[sspec, cspec, sspec, cspec, wspec,
                  sspec, cspec, sspec, cspec, wspec],
        out_specs=[pl.BlockSpec((BR, H), lambda i: (i, 0))] * 2,
        out_shape=[jax.ShapeDtypeStruct((N, H), jnp.float32)] * 2,
    )(sumsA, cntsA, sumsB, cntsB, W1, sumsC, cntsC, sumsD, cntsD, W3)


def kernel(features, W1, W3, source_nei, target_nei, source_nei2, target_nei2):
    def prep(nei):
        return nei[1], nei[0]

    srcA, dstA = prep(source_nei)
    srcB, dstB = prep(target_nei2)
    srcC, dstC = prep(target_nei)
    srcD, dstD = prep(source_nei2)

    zrows = jnp.zeros((RPS, D), jnp.float32)
    zcnt = jnp.zeros((RPS, CW), jnp.float32)
    ones_h = jnp.ones((K, CW), jnp.float32)

    (sumsA, cntsA, sumsB, cntsB,
     sumsC, cntsC, sumsD, cntsD) = _sc_aggregate(
        features, srcA, dstA, srcB, dstB, srcC, dstC, srcD, dstD,
        zrows, zcnt, ones_h)

    return tuple(_tc_finish(sumsA, cntsA, sumsB, cntsB, W1,
                            sumsC, cntsC, sumsD, cntsD, W3))
